# baseline (speedup 1.0000x reference)
import functools

import jax
import jax.numpy as jnp
from jax import lax
from jax.experimental import pallas as pl
from jax.experimental.pallas import tpu as pltpu
from jax.experimental.pallas import tpu_sc as plsc

B, C, N, K, H = 1, 256, 10000, 16, 1
OUT = 256
NE = N * K
NC, NS = 2, 16
NW = NC * NS

P = 1
N_P = N // P
NE_P = NE // P
EPW = NE_P // NW
R = 40
NCHUNK = EPW // R

NB = 400
GRID_P = N_P // NB


QG = 4
QD = 2


def _sc_lo_f32(v):
    return lax.bitcast_convert_type(v << 16, jnp.float32)


def _sc_hi_f32(v):
    return lax.bitcast_convert_type((v >> 16) << 16, jnp.float32)


def _make_sc_gather_diff(part):
    mesh = plsc.VectorSubcoreMesh(
        core_axis_name="c", subcore_axis_name="s", num_cores=NC,
        num_subcores=NS)

    @functools.partial(
        pl.kernel,
        out_type=jax.ShapeDtypeStruct((NE_P, C // 2), jnp.int32),
        mesh=mesh,
        scratch_types=[
            pltpu.VMEM((EPW,), jnp.int32),
            pltpu.VMEM((EPW,), jnp.int32),
            [pltpu.VMEM((R, C // 2), jnp.int32)] * QG,
            [pltpu.VMEM((R, C // 2), jnp.int32)] * QG,
            [pltpu.VMEM((R, C // 2), jnp.int32)] * QD,
            [pltpu.SemaphoreType.DMA] * QG,
            [pltpu.SemaphoreType.DMA] * QD,
        ],
    )
    def k(xt_hbm, ej_hbm, ei_hbm, d_hbm, idxj_all, idxi_all, bjs, bis,
          bds, gsems, wsems):
        wid = lax.axis_index("s") * NC + lax.axis_index("c")
        src_base = part * NE_P + wid * EPW
        dst_base = wid * EPW
        pltpu.sync_copy(ej_hbm.at[pl.ds(src_base, EPW)], idxj_all)
        pltpu.sync_copy(ei_hbm.at[pl.ds(src_base, EPW)], idxi_all)

        def start(t, s):
            off = t * R
            pltpu.async_copy(
                xt_hbm.at[idxj_all.at[pl.ds(off, R)]], bjs[s], gsems[s])
            pltpu.async_copy(
                xt_hbm.at[idxi_all.at[pl.ds(off, R)]], bis[s], gsems[s])

        def wait_gather(s):
            pltpu.make_async_copy(
                xt_hbm.at[idxj_all.at[pl.ds(0, R)]], bjs[s],
                gsems[s]).wait()
            pltpu.make_async_copy(
                xt_hbm.at[idxi_all.at[pl.ds(0, R)]], bis[s],
                gsems[s]).wait()

        def wait_wb(sd):
            pltpu.make_async_copy(
                bds[sd], d_hbm.at[pl.ds(dst_base, R)], wsems[sd]).wait()

        for t in range(QG):
            start(t, t)

        def step(t, sg, sd):
            wait_gather(sg)

            @pl.when(t >= QD)
            def _():
                wait_wb(sd)

            bj, bi, bd = bjs[sg], bis[sg], bds[sd]

            def rowloop(r, c2):
                for cc in range(C // 32):
                    sl = pl.ds(cc * 16, 16)
                    a = bj[r, sl]
                    bb = bi[r, sl]
                    de = _sc_lo_f32(a) - _sc_lo_f32(bb)
                    do = _sc_hi_f32(a) - _sc_hi_f32(bb)
                    der = lax.bitcast_convert_type(de, jnp.int32) + 32768
                    dor = lax.bitcast_convert_type(do, jnp.int32) + 32768
                    bd[r, sl] = (
                        lax.shift_right_logical(der, 16)
                        | (dor & jnp.int32(-65536))
                    )
                return c2

            lax.fori_loop(0, R, rowloop, 0)
            pltpu.async_copy(bd, d_hbm.at[pl.ds(dst_base + t * R, R)],
                             wsems[sd])

            @pl.when(t + QG < NCHUNK)
            def _():
                start(t + QG, sg)

        def body(g, carry):
            for u in range(QG):
                t = QG * g + u

                @pl.when(t < NCHUNK)
                def _():
                    step(t, u, u % QD)
            return carry

        lax.fori_loop(0, (NCHUNK + QG - 1) // QG, body, 0)
        for t in range(NCHUNK - QD, NCHUNK):
            wait_wb(t % QD)

    return k


def _lo_f32(x32):
    return lax.bitcast_convert_type(x32 << 16, jnp.float32)


def _hi_f32(x32):
    return lax.bitcast_convert_type((x32 >> 16) << 16, jnp.float32)


def _tc_body(d_ref, xt_ref, atte_ref, atto_ref, wt1_ref,
             wt2e_ref, wt2o_ref, b_ref, out_ref):
    d32 = d_ref[...]
    d3e = _lo_f32(d32).reshape(NB, K, C // 2)
    d3o = _hi_f32(d32).reshape(NB, K, C // 2)
    acte = jnp.where(d3e >= 0, d3e, 0.2 * d3e)
    acto = jnp.where(d3o >= 0, d3o, 0.2 * d3o)
    logits = (
        jnp.sum(acte * atte_ref[...][None, :, :], axis=2)
        + jnp.sum(acto * atto_ref[...][None, :, :], axis=2)
    )
    m = jnp.max(logits, axis=1, keepdims=True)
    e = jnp.exp(logits - m)
    alpha = e / jnp.sum(e, axis=1, keepdims=True)
    aw = alpha[:, :, None]
    xmaxe = jnp.max(d3e * aw, axis=1).astype(jnp.bfloat16)
    xmaxo = jnp.max(d3o * aw, axis=1).astype(jnp.bfloat16)
    xt = xt_ref[...]
    y = (
        jnp.dot(xt, wt1_ref[...], preferred_element_type=jnp.float32)
        + jnp.dot(xmaxe, wt2e_ref[...], preferred_element_type=jnp.float32)
        + jnp.dot(xmaxo, wt2o_ref[...], preferred_element_type=jnp.float32)
        + b_ref[...]
    )
    out_ref[...] = y


def _tc_part(part, d32, xt, atte, atto, wt1, wt2e, wt2o, b2):
    return pl.pallas_call(
        _tc_body,
        grid=(GRID_P,),
        in_specs=[
            pl.BlockSpec((NB * K, C // 2), lambda i: (i, 0)),
            pl.BlockSpec((NB, C), lambda i, p=part: (i + p * GRID_P, 0)),
            pl.BlockSpec((1, C // 2), lambda i: (0, 0)),
            pl.BlockSpec((1, C // 2), lambda i: (0, 0)),
            pl.BlockSpec((C, OUT), lambda i: (0, 0)),
            pl.BlockSpec((C // 2, OUT), lambda i: (0, 0)),
            pl.BlockSpec((C // 2, OUT), lambda i: (0, 0)),
            pl.BlockSpec((1, OUT), lambda i: (0, 0)),
        ],
        out_specs=pl.BlockSpec((NB, OUT), lambda i: (i, 0)),
        out_shape=jax.ShapeDtypeStruct((N_P, OUT), jnp.float32),
    )(d32, xt, atte, atto, wt1, wt2e, wt2o, b2)


def kernel(x, edge_index, att, W, b):
    xt = jnp.transpose(x[0, :, :, 0])
    xbits = lax.bitcast_convert_type(xt, jnp.int32)
    lo = xbits[:, : C // 2] + 32768
    hi = xbits[:, C // 2:] + 32768
    xt32 = lax.shift_right_logical(lo, 16) | (hi & jnp.int32(-65536))
    ej = edge_index[0, 0].reshape(NE)
    ei = edge_index[1, 0].reshape(NE)
    att_v = att.reshape(1, C)
    atte = att_v[:, : C // 2]
    atto = att_v[:, C // 2:]
    wt = jnp.transpose(W)
    wt1 = wt[:C].astype(jnp.bfloat16)
    wt2e = wt[C:C + C // 2].astype(jnp.bfloat16)
    wt2o = wt[C + C // 2:].astype(jnp.bfloat16)
    xtb = xt.astype(jnp.bfloat16)
    b2 = b.reshape(1, OUT)

    ds32 = [_make_sc_gather_diff(p)(xt32, ej, ei) for p in range(P)]
    yts = [
        _tc_part(p, ds32[p], xtb, atte, atto, wt1, wt2e, wt2o, b2)
        for p in range(P)
    ]
    yt = jnp.concatenate(yts, axis=0)
    return jnp.transpose(yt)[None, :, :, None]


# speedup vs baseline: 1.0231x; 1.0231x over previous
import functools

import jax
import jax.numpy as jnp
from jax import lax
from jax.experimental import pallas as pl
from jax.experimental.pallas import tpu as pltpu
from jax.experimental.pallas import tpu_sc as plsc

B, C, N, K, H = 1, 256, 10000, 16, 1
OUT = 256
NE = N * K
NC, NS = 2, 16
NW = NC * NS

P = 1
N_P = N // P
NE_P = NE // P
EPW = NE_P // NW
R = 40
NCHUNK = EPW // R

NB = 400
GRID_P = N_P // NB


QG = 6
QD = 2


def _sc_lo_f32(v):
    return lax.bitcast_convert_type(v << 16, jnp.float32)


def _sc_hi_f32(v):
    return lax.bitcast_convert_type((v >> 16) << 16, jnp.float32)


def _make_sc_gather_diff(part):
    mesh = plsc.VectorSubcoreMesh(
        core_axis_name="c", subcore_axis_name="s", num_cores=NC,
        num_subcores=NS)

    @functools.partial(
        pl.kernel,
        out_type=jax.ShapeDtypeStruct((NE_P, C // 2), jnp.int32),
        mesh=mesh,
        scratch_types=[
            pltpu.VMEM((EPW,), jnp.int32),
            pltpu.VMEM((EPW,), jnp.int32),
            [pltpu.VMEM((R, C // 2), jnp.int32)] * QG,
            [pltpu.VMEM((R, C // 2), jnp.int32)] * QG,
            [pltpu.VMEM((R, C // 2), jnp.int32)] * QD,
            [pltpu.SemaphoreType.DMA] * QG,
            [pltpu.SemaphoreType.DMA] * QD,
        ],
    )
    def k(xt_hbm, ej_hbm, ei_hbm, d_hbm, idxj_all, idxi_all, bjs, bis,
          bds, gsems, wsems):
        wid = lax.axis_index("s") * NC + lax.axis_index("c")
        src_base = part * NE_P + wid * EPW
        dst_base = wid * EPW
        pltpu.sync_copy(ej_hbm.at[pl.ds(src_base, EPW)], idxj_all)
        pltpu.sync_copy(ei_hbm.at[pl.ds(src_base, EPW)], idxi_all)

        def start(t, s):
            off = t * R
            pltpu.async_copy(
                xt_hbm.at[idxj_all.at[pl.ds(off, R)]], bjs[s], gsems[s])
            pltpu.async_copy(
                xt_hbm.at[idxi_all.at[pl.ds(off, R)]], bis[s], gsems[s])

        def wait_gather(s):
            pltpu.make_async_copy(
                xt_hbm.at[idxj_all.at[pl.ds(0, R)]], bjs[s],
                gsems[s]).wait()
            pltpu.make_async_copy(
                xt_hbm.at[idxi_all.at[pl.ds(0, R)]], bis[s],
                gsems[s]).wait()

        def wait_wb(sd):
            pltpu.make_async_copy(
                bds[sd], d_hbm.at[pl.ds(dst_base, R)], wsems[sd]).wait()

        for t in range(QG):
            start(t, t)

        def step(t, sg, sd):
            wait_gather(sg)

            @pl.when(t >= QD)
            def _():
                wait_wb(sd)

            bj, bi, bd = bjs[sg], bis[sg], bds[sd]

            def rowloop(r, c2):
                for cc in range(C // 32):
                    sl = pl.ds(cc * 16, 16)
                    a = bj[r, sl]
                    bb = bi[r, sl]
                    de = _sc_lo_f32(a) - _sc_lo_f32(bb)
                    do = _sc_hi_f32(a) - _sc_hi_f32(bb)
                    der = lax.bitcast_convert_type(de, jnp.int32) + 32768
                    dor = lax.bitcast_convert_type(do, jnp.int32) + 32768
                    bd[r, sl] = (
                        lax.shift_right_logical(der, 16)
                        | (dor & jnp.int32(-65536))
                    )
                return c2

            lax.fori_loop(0, R, rowloop, 0)
            pltpu.async_copy(bd, d_hbm.at[pl.ds(dst_base + t * R, R)],
                             wsems[sd])

            @pl.when(t + QG < NCHUNK)
            def _():
                start(t + QG, sg)

        def body(g, carry):
            for u in range(QG):
                t = QG * g + u

                @pl.when(t < NCHUNK)
                def _():
                    step(t, u, u % QD)
            return carry

        lax.fori_loop(0, (NCHUNK + QG - 1) // QG, body, 0)
        for t in range(NCHUNK - QD, NCHUNK):
            wait_wb(t % QD)

    return k


def _lo_f32(x32):
    return lax.bitcast_convert_type(x32 << 16, jnp.float32)


def _hi_f32(x32):
    return lax.bitcast_convert_type((x32 >> 16) << 16, jnp.float32)


def _tc_body(d_ref, xt_ref, atte_ref, atto_ref, wt1_ref,
             wt2e_ref, wt2o_ref, b_ref, out_ref):
    d32 = d_ref[...]
    d3e = _lo_f32(d32).reshape(NB, K, C // 2)
    d3o = _hi_f32(d32).reshape(NB, K, C // 2)
    acte = jnp.where(d3e >= 0, d3e, 0.2 * d3e)
    acto = jnp.where(d3o >= 0, d3o, 0.2 * d3o)
    logits = (
        jnp.sum(acte * atte_ref[...][None, :, :], axis=2)
        + jnp.sum(acto * atto_ref[...][None, :, :], axis=2)
    )
    m = jnp.max(logits, axis=1, keepdims=True)
    e = jnp.exp(logits - m)
    alpha = e / jnp.sum(e, axis=1, keepdims=True)
    aw = alpha[:, :, None]
    xmaxe = jnp.max(d3e * aw, axis=1)
    xmaxo = jnp.max(d3o * aw, axis=1)
    xt = xt_ref[...]
    y = (
        jnp.dot(xt, wt1_ref[...], preferred_element_type=jnp.float32)
        + jnp.dot(xmaxe, wt2e_ref[...], preferred_element_type=jnp.float32)
        + jnp.dot(xmaxo, wt2o_ref[...], preferred_element_type=jnp.float32)
        + b_ref[...]
    )
    out_ref[...] = y


def _tc_part(part, d32, xt, atte, atto, wt1, wt2e, wt2o, b2):
    return pl.pallas_call(
        _tc_body,
        grid=(GRID_P,),
        in_specs=[
            pl.BlockSpec((NB * K, C // 2), lambda i: (i, 0)),
            pl.BlockSpec((NB, C), lambda i, p=part: (i + p * GRID_P, 0)),
            pl.BlockSpec((1, C // 2), lambda i: (0, 0)),
            pl.BlockSpec((1, C // 2), lambda i: (0, 0)),
            pl.BlockSpec((C, OUT), lambda i: (0, 0)),
            pl.BlockSpec((C // 2, OUT), lambda i: (0, 0)),
            pl.BlockSpec((C // 2, OUT), lambda i: (0, 0)),
            pl.BlockSpec((1, OUT), lambda i: (0, 0)),
        ],
        out_specs=pl.BlockSpec((NB, OUT), lambda i: (i, 0)),
        out_shape=jax.ShapeDtypeStruct((N_P, OUT), jnp.float32),
    )(d32, xt, atte, atto, wt1, wt2e, wt2o, b2)


def kernel(x, edge_index, att, W, b):
    xt = jnp.transpose(x[0, :, :, 0])
    xbits = lax.bitcast_convert_type(xt, jnp.int32)
    lo = xbits[:, : C // 2] + 32768
    hi = xbits[:, C // 2:] + 32768
    xt32 = lax.shift_right_logical(lo, 16) | (hi & jnp.int32(-65536))
    ej = edge_index[0, 0].reshape(NE)
    ei = edge_index[1, 0].reshape(NE)
    att_v = att.reshape(1, C)
    atte = att_v[:, : C // 2]
    atto = att_v[:, C // 2:]
    wt = jnp.transpose(W)
    wt1 = wt[:C]
    wt2e = wt[C:C + C // 2]
    wt2o = wt[C + C // 2:]
    b2 = b.reshape(1, OUT)

    ds32 = [_make_sc_gather_diff(p)(xt32, ej, ei) for p in range(P)]
    yts = [
        _tc_part(p, ds32[p], xt, atte, atto, wt1, wt2e, wt2o, b2)
        for p in range(P)
    ]
    yt = jnp.concatenate(yts, axis=0)
    return jnp.transpose(yt)[None, :, :, None]
